# Initial kernel scaffold; baseline (speedup 1.0000x reference)
import jax
import jax.numpy as jnp
from jax.experimental import pallas as pl


def kernel(weight, edge_index, W_lin, b_lin, W0, b0, W1, b1, W2, b2, Wp, bp):
    raise NotImplementedError("write your pallas kernel here")


# speedup vs baseline: 10.9712x; 10.9712x over previous
import functools

import jax
import jax.numpy as jnp
from jax import lax
from jax.experimental import pallas as pl
from jax.experimental.pallas import tpu as pltpu, tpu_sc as plsc

N = 10000
D = 256
E = 160000

NPAD = 10240
NS = 16
L = 16
EPW = E // NS
G = 80
NCH = EPW // G
HALF = NPAD // 2
PERT = HALF // NS
TRASH = 64
ACC = HALF + TRASH
BR = 256
NBLK = NPAD // BR

_SLOPE = 0.01


def _mesh():
    return plsc.VectorSubcoreMesh(core_axis_name="c", subcore_axis_name="s")


def _rsqrt16(x):
    i = plsc.bitcast(x, jnp.int32)
    i = jnp.int32(0x5F3759DF) - lax.shift_right_logical(i, 1)
    y = plsc.bitcast(i, jnp.float32)
    for _ in range(3):
        y = y * (1.5 - 0.5 * x * y * y)
    return y


def _build_sidx(node_v, sidx_v, base):
    def body(r, _):
        for cc in range(G // L):
            d = node_v[pl.ds(r * G + cc * L, L)]
            local = d - base
            inr = (local >= 0) & (local < HALF)
            trash = HALF + lax.rem(d, jnp.full((L,), TRASH, jnp.int32))
            sidx_v[r, pl.ds(cc * L, L)] = lax.select(inr, local, trash)
        return 0

    lax.fori_loop(0, NCH, body, 0)


def _k1_call(src, dst, weight_pad):
    @functools.partial(
        pl.kernel,
        out_type=(
            jax.ShapeDtypeStruct((NPAD,), jnp.float32),
            jax.ShapeDtypeStruct((NPAD,), jnp.float32),
            jax.ShapeDtypeStruct((NPAD,), jnp.float32),
        ),
        mesh=_mesh(),
        compiler_params=pltpu.CompilerParams(needs_layout_passes=False),
        scratch_types=[
            pltpu.VMEM((EPW,), jnp.int32),
            pltpu.VMEM((EPW,), jnp.int32),
            pltpu.VMEM((NCH, G), jnp.int32),
            pltpu.VMEM((NCH, G), jnp.int32),
            pltpu.VMEM((G,), jnp.float32),
            pltpu.VMEM((PERT,), jnp.float32),
            pltpu.VMEM((PERT,), jnp.float32),
            pltpu.VMEM((PERT,), jnp.float32),
            pltpu.VMEM_SHARED((ACC,), jnp.float32),
            pltpu.VMEM_SHARED((ACC,), jnp.float32),
        ],
    )
    def k1(src_hbm, dst_hbm, w_hbm, no_hbm, ni_hbm, nw_hbm,
           src_v, dst_v, sxi_v, sdi_v, ones_v, sa_v, sb_v, sc_v,
           dego_sh, degi_sh):
        cid = lax.axis_index("c")
        sid = lax.axis_index("s")
        base = cid * HALF

        pltpu.sync_copy(src_hbm.at[pl.ds(sid * EPW, EPW)], src_v)
        pltpu.sync_copy(dst_hbm.at[pl.ds(sid * EPW, EPW)], dst_v)
        _build_sidx(src_v, sxi_v, base)
        _build_sidx(dst_v, sdi_v, base)

        for i in range(G // L):
            ones_v[pl.ds(i * L, L)] = jnp.full((L,), 1.0, jnp.float32)

        def initb(i, _):
            sa_v[pl.ds(i * L, L)] = jnp.full((L,), 1.0, jnp.float32)
            return 0

        lax.fori_loop(0, PERT // L, initb, 0)
        pltpu.sync_copy(sa_v, dego_sh.at[pl.ds(sid * PERT, PERT)])
        pltpu.sync_copy(sa_v, degi_sh.at[pl.ds(sid * PERT, PERT)])
        plsc.subcore_barrier()

        def chunk(j, _):
            pltpu.sync_copy(ones_v, dego_sh.at[sxi_v.at[j]], add=True)
            pltpu.sync_copy(ones_v, degi_sh.at[sdi_v.at[j]], add=True)
            return 0

        lax.fori_loop(0, NCH, chunk, 0)
        plsc.subcore_barrier()

        pltpu.sync_copy(dego_sh.at[pl.ds(sid * PERT, PERT)], sa_v)
        pltpu.sync_copy(degi_sh.at[pl.ds(sid * PERT, PERT)], sb_v)
        pltpu.sync_copy(w_hbm.at[pl.ds(base + sid * PERT, PERT)], sc_v)

        def normb(i, _):
            sl = pl.ds(i * L, L)
            no = _rsqrt16(sa_v[sl])
            ni = _rsqrt16(sb_v[sl])
            w = sc_v[sl]
            sa_v[sl] = no
            sb_v[sl] = ni
            sc_v[sl] = no * w
            return 0

        lax.fori_loop(0, PERT // L, normb, 0)
        pltpu.sync_copy(sa_v, no_hbm.at[pl.ds(base + sid * PERT, PERT)])
        pltpu.sync_copy(sb_v, ni_hbm.at[pl.ds(base + sid * PERT, PERT)])
        pltpu.sync_copy(sc_v, nw_hbm.at[pl.ds(base + sid * PERT, PERT)])

    return k1(src, dst, weight_pad)


def _k3_call(src, dst, nw, no):
    @functools.partial(
        pl.kernel,
        out_type=(
            jax.ShapeDtypeStruct((NPAD,), jnp.float32),
            jax.ShapeDtypeStruct((NPAD,), jnp.float32),
        ),
        mesh=_mesh(),
        compiler_params=pltpu.CompilerParams(needs_layout_passes=False),
        scratch_types=[
            pltpu.VMEM((EPW,), jnp.int32),
            pltpu.VMEM((EPW,), jnp.int32),
            pltpu.VMEM((NCH, G), jnp.int32),
            pltpu.VMEM((G,), jnp.float32),
            pltpu.VMEM((G,), jnp.float32),
            pltpu.VMEM((NPAD // NS,), jnp.float32),
            pltpu.VMEM_SHARED((NPAD,), jnp.float32),
            pltpu.VMEM_SHARED((NPAD,), jnp.float32),
            pltpu.VMEM_SHARED((ACC,), jnp.float32),
            pltpu.VMEM_SHARED((ACC,), jnp.float32),
            pltpu.SemaphoreType.DMA,
            pltpu.SemaphoreType.DMA,
        ],
    )
    def k3(src_hbm, dst_hbm, nw_hbm, no_hbm, u_hbm, v_hbm,
           src_v, dst_v, sdi_v, va_v, vb_v, stage_v, nwt_sh, not_sh,
           uacc_sh, vacc_sh, sem1, sem2):
        cid = lax.axis_index("c")
        sid = lax.axis_index("s")
        base = cid * HALF

        pltpu.sync_copy(src_hbm.at[pl.ds(sid * EPW, EPW)], src_v)
        pltpu.sync_copy(dst_hbm.at[pl.ds(sid * EPW, EPW)], dst_v)
        _build_sidx(dst_v, sdi_v, base)

        TPW = NPAD // NS
        tsl = pl.ds(sid * TPW, TPW)
        pltpu.sync_copy(nw_hbm.at[tsl], stage_v)
        pltpu.sync_copy(stage_v, nwt_sh.at[tsl])
        pltpu.sync_copy(no_hbm.at[tsl], stage_v)
        pltpu.sync_copy(stage_v, not_sh.at[tsl])
        hsl = pl.ds(base + sid * PERT, PERT)
        asl = pl.ds(sid * PERT, PERT)
        st = stage_v.at[pl.ds(0, PERT)]
        pltpu.sync_copy(nw_hbm.at[hsl], st)
        pltpu.sync_copy(st, uacc_sh.at[asl])
        pltpu.sync_copy(no_hbm.at[hsl], st)
        pltpu.sync_copy(st, vacc_sh.at[asl])
        plsc.subcore_barrier()

        def chunk(j, _):
            g = pl.ds(j * G, G)
            cp1 = pltpu.async_copy(nwt_sh.at[src_v.at[g]], va_v, sem1)
            cp2 = pltpu.async_copy(not_sh.at[src_v.at[g]], vb_v, sem2)
            cp1.wait()
            cp2.wait()
            pltpu.sync_copy(va_v, uacc_sh.at[sdi_v.at[j]], add=True)
            pltpu.sync_copy(vb_v, vacc_sh.at[sdi_v.at[j]], add=True)
            return 0

        lax.fori_loop(0, NCH, chunk, 0)
        plsc.subcore_barrier()

        pltpu.sync_copy(uacc_sh.at[asl], st)
        pltpu.sync_copy(st, u_hbm.at[hsl])
        pltpu.sync_copy(vacc_sh.at[asl], st)
        pltpu.sync_copy(st, v_hbm.at[hsl])

    return k3(src, dst, nw, no)


def _k4_call(u2, v2, ni2, no2, W_lin, b_lin2, W0, b02):
    def body(u_ref, v_ref, ni_ref, no_ref, wl_ref, bl_ref, w0_ref, b0_ref,
             m_ref):
        r1 = jnp.dot(wl_ref[...], w0_ref[...],
                     preferred_element_type=jnp.float32)
        r2 = jnp.dot(bl_ref[...], w0_ref[...],
                     preferred_element_type=jnp.float32)
        ni = ni_ref[...]
        au = ni * u_ref[...]
        av = ni * v_ref[...]
        h0 = au * r1 + av * r2 + b0_ref[...]
        g0 = jnp.where(h0 >= 0, h0, _SLOPE * h0)
        m_ref[...] = g0 * no_ref[...]

    vec = pl.BlockSpec((BR, 1), lambda i: (i, 0))
    full = lambda s: pl.BlockSpec(s, lambda i: (0, 0))
    return pl.pallas_call(
        body,
        grid=(NBLK,),
        in_specs=[vec, vec, vec, vec, full((1, D)), full((1, D)),
                  full((D, D)), full((1, D))],
        out_specs=pl.BlockSpec((BR, D), lambda i: (i, 0)),
        out_shape=jax.ShapeDtypeStruct((NPAD, D), jnp.float32),
    )(u2, v2, ni2, no2, W_lin, b_lin2, W0, b02)


DH = D // 2


def _k5_call(m, src, dst):
    @functools.partial(
        pl.kernel,
        out_type=jax.ShapeDtypeStruct((NPAD, DH), jnp.float32),
        mesh=_mesh(),
        compiler_params=pltpu.CompilerParams(needs_layout_passes=False),
        scratch_types=[
            pltpu.VMEM((EPW,), jnp.int32),
            pltpu.VMEM((EPW,), jnp.int32),
            pltpu.VMEM((NCH, G), jnp.int32),
            pltpu.VMEM((G, DH), jnp.float32),
            pltpu.VMEM((G, DH), jnp.float32),
            pltpu.VMEM_SHARED((ACC, DH), jnp.float32),
            pltpu.SemaphoreType.DMA,
            pltpu.SemaphoreType.DMA,
        ],
    )
    def k5(m_hbm, src_hbm, dst_hbm, agg_hbm,
           src_v, dst_v, sdi_v, rows0_v, rows1_v, acc_sh, sem0, sem1):
        cid = lax.axis_index("c")
        sid = lax.axis_index("s")
        base = cid * HALF

        pltpu.sync_copy(src_hbm.at[pl.ds(sid * EPW, EPW)], src_v)
        pltpu.sync_copy(dst_hbm.at[pl.ds(sid * EPW, EPW)], dst_v)
        _build_sidx(dst_v, sdi_v, base)

        for p in range(PERT // G):
            pltpu.sync_copy(
                m_hbm.at[pl.ds(base + sid * PERT + p * G, G)], rows0_v)
            pltpu.sync_copy(rows0_v, acc_sh.at[pl.ds(sid * PERT + p * G, G)])
        plsc.subcore_barrier()

        cp0 = pltpu.async_copy(m_hbm.at[src_v.at[pl.ds(0, G)]], rows0_v, sem0)

        def pair(i, _):
            j0 = 2 * i
            j1 = 2 * i + 1
            cpb = pltpu.async_copy(
                m_hbm.at[src_v.at[pl.ds(j1 * G, G)]], rows1_v, sem1)
            pltpu.make_async_copy(
                m_hbm.at[src_v.at[pl.ds(j0 * G, G)]], rows0_v, sem0).wait()
            pltpu.sync_copy(rows0_v, acc_sh.at[sdi_v.at[j0]], add=True)
            cpa = pltpu.async_copy(
                m_hbm.at[src_v.at[pl.ds((j1 + 1) * G, G)]], rows0_v, sem0)
            cpb.wait()
            pltpu.sync_copy(rows1_v, acc_sh.at[sdi_v.at[j1]], add=True)
            return 0

        lax.fori_loop(0, (NCH - 1) // 2, pair, 0)
        pltpu.make_async_copy(
            m_hbm.at[src_v.at[pl.ds((NCH - 1) * G, G)]], rows0_v, sem0).wait()
        pltpu.sync_copy(rows0_v, acc_sh.at[sdi_v.at[NCH - 1]], add=True)
        plsc.subcore_barrier()

        for p in range(PERT // G):
            pltpu.sync_copy(acc_sh.at[pl.ds(sid * PERT + p * G, G)], rows0_v)
            pltpu.sync_copy(
                rows0_v, agg_hbm.at[pl.ds(base + sid * PERT + p * G, G)])

    return k5(m, src, dst)


def _k6_call(agg, ni2, no2, W1, b12, W2, Wp, b22, bp2):
    def body(agg_ref, ni_ref, no_ref, w1_ref, b1_ref, w2_ref, wp_ref,
             b2_ref, bp_ref, t_ref, c0_ref):
        x = agg_ref[...] * ni_ref[...]
        h1 = jnp.dot(x, w1_ref[...],
                     preferred_element_type=jnp.float32) + b1_ref[...]
        g1 = jnp.where(h1 >= 0, h1, _SLOPE * h1)
        w2p = jnp.dot(w2_ref[...], wp_ref[...],
                      preferred_element_type=jnp.float32)
        q = jnp.dot(g1, w2p, preferred_element_type=jnp.float32)
        t_ref[...] = no_ref[...] * q

        @pl.when(pl.program_id(0) == 0)
        def _():
            c0 = (jnp.dot(b2_ref[...], wp_ref[...],
                          preferred_element_type=jnp.float32)[0, 0]
                  + bp_ref[0, 0])
            c0_ref[...] = jnp.full((1, L), c0, jnp.float32)

    vec = pl.BlockSpec((BR, 1), lambda i: (i, 0))
    full = lambda s: pl.BlockSpec(s, lambda i: (0, 0))
    return pl.pallas_call(
        body,
        grid=(NBLK,),
        in_specs=[pl.BlockSpec((BR, D), lambda i: (i, 0)), vec, vec,
                  full((D, D)), full((1, D)), full((D, D)), full((D, 1)),
                  full((1, D)), full((1, 1))],
        out_specs=[vec, full((1, L))],
        out_shape=[jax.ShapeDtypeStruct((NPAD, 1), jnp.float32),
                   jax.ShapeDtypeStruct((1, L), jnp.float32)],
    )(agg, ni2, no2, W1, b12, W2, Wp, b22, bp2)


def _k7_call(t, src, dst, ni, c0b):
    @functools.partial(
        pl.kernel,
        out_type=jax.ShapeDtypeStruct((NPAD,), jnp.float32),
        mesh=_mesh(),
        compiler_params=pltpu.CompilerParams(needs_layout_passes=False),
        scratch_types=[
            pltpu.VMEM((EPW,), jnp.int32),
            pltpu.VMEM((EPW,), jnp.int32),
            pltpu.VMEM((NCH, G), jnp.int32),
            pltpu.VMEM((G,), jnp.float32),
            pltpu.VMEM((PERT,), jnp.float32),
            pltpu.VMEM((PERT,), jnp.float32),
            pltpu.VMEM((L,), jnp.float32),
            pltpu.VMEM((NPAD // NS,), jnp.float32),
            pltpu.VMEM_SHARED((NPAD,), jnp.float32),
            pltpu.VMEM_SHARED((ACC,), jnp.float32),
            pltpu.SemaphoreType.DMA,
        ],
    )
    def k7(t_hbm, src_hbm, dst_hbm, ni_hbm, c0_hbm, out_hbm,
           src_v, dst_v, sdi_v, va_v, sa_v, sb_v, c0_v, stage_v, tt_sh,
           sacc_sh, sem1):
        cid = lax.axis_index("c")
        sid = lax.axis_index("s")
        base = cid * HALF

        pltpu.sync_copy(src_hbm.at[pl.ds(sid * EPW, EPW)], src_v)
        pltpu.sync_copy(dst_hbm.at[pl.ds(sid * EPW, EPW)], dst_v)
        _build_sidx(dst_v, sdi_v, base)

        TPW = NPAD // NS
        tsl = pl.ds(sid * TPW, TPW)
        pltpu.sync_copy(t_hbm.at[tsl], stage_v)
        pltpu.sync_copy(stage_v, tt_sh.at[tsl])
        hsl = pl.ds(base + sid * PERT, PERT)
        asl = pl.ds(sid * PERT, PERT)
        pltpu.sync_copy(t_hbm.at[hsl], sa_v)
        pltpu.sync_copy(sa_v, sacc_sh.at[asl])
        plsc.subcore_barrier()

        def chunk(j, _):
            pltpu.async_copy(
                tt_sh.at[src_v.at[pl.ds(j * G, G)]], va_v, sem1).wait()
            pltpu.sync_copy(va_v, sacc_sh.at[sdi_v.at[j]], add=True)
            return 0

        lax.fori_loop(0, NCH, chunk, 0)
        plsc.subcore_barrier()

        pltpu.sync_copy(sacc_sh.at[asl], sa_v)
        pltpu.sync_copy(ni_hbm.at[hsl], sb_v)
        pltpu.sync_copy(c0_hbm, c0_v)

        def outb(i, _):
            sl = pl.ds(i * L, L)
            sa_v[sl] = sb_v[sl] * sa_v[sl] + c0_v[...]
            return 0

        lax.fori_loop(0, PERT // L, outb, 0)
        pltpu.sync_copy(sa_v, out_hbm.at[hsl])

    return k7(t, src, dst, ni, c0b)


def kernel(weight, edge_index, W_lin, b_lin, W0, b0, W1, b1, W2, b2, Wp, bp):
    src = edge_index[0]
    dst = edge_index[1]
    weight_pad = jnp.pad(weight, (0, NPAD - N))

    no, ni, nw = _k1_call(src, dst, weight_pad)
    u, v = _k3_call(src, dst, nw, no)

    to2 = lambda a: a.reshape(NPAD, 1)
    m = _k4_call(to2(u), to2(v), to2(ni), to2(no),
                 W_lin, b_lin.reshape(1, D), W0, b0.reshape(1, D))
    agg = jnp.concatenate(
        [_k5_call(m[:, :DH], src, dst), _k5_call(m[:, DH:], src, dst)],
        axis=1)
    t2, c0b = _k6_call(agg, to2(ni), to2(no), W1, b1.reshape(1, D),
                       W2, Wp, b2.reshape(1, D), bp.reshape(1, 1))
    logits_flat = _k7_call(t2.reshape(NPAD), src, dst, ni, c0b.reshape(L))
    return logits_flat[:N, None]


# speedup vs baseline: 12.8456x; 1.1709x over previous
import functools

import jax
import jax.numpy as jnp
from jax import lax
from jax.experimental import pallas as pl
from jax.experimental.pallas import tpu as pltpu, tpu_sc as plsc

N = 10000
D = 256
E = 160000

NPAD = 10240
NS = 16
L = 16
ECORE = E // 2
EPT = ECORE // NS
G = 40
NCH = EPT // G
TPW = NPAD // NS
DH = D // 2
BR = 256
NBLK = NPAD // BR

_SLOPE = 0.01


def _mesh():
    return plsc.VectorSubcoreMesh(core_axis_name="c", subcore_axis_name="s")


def _rsqrt16(x):
    i = plsc.bitcast(x, jnp.int32)
    i = jnp.int32(0x5F3759DF) - lax.shift_right_logical(i, 1)
    y = plsc.bitcast(i, jnp.float32)
    for _ in range(3):
        y = y * (1.5 - 0.5 * x * y * y)
    return y


def _k1_call(src, dst):
    @functools.partial(
        pl.kernel,
        out_type=(
            jax.ShapeDtypeStruct((2 * NPAD,), jnp.float32),
            jax.ShapeDtypeStruct((2 * NPAD,), jnp.float32),
        ),
        mesh=_mesh(),
        compiler_params=pltpu.CompilerParams(needs_layout_passes=False),
        scratch_types=[
            pltpu.VMEM((EPT,), jnp.int32),
            pltpu.VMEM((EPT,), jnp.int32),
            pltpu.VMEM((G,), jnp.float32),
            pltpu.VMEM((TPW,), jnp.float32),
            pltpu.VMEM_SHARED((NPAD,), jnp.float32),
            pltpu.VMEM_SHARED((NPAD,), jnp.float32),
        ],
    )
    def k1(src_hbm, dst_hbm, do_hbm, di_hbm,
           src_v, dst_v, ones_v, z_v, dego_sh, degi_sh):
        cid = lax.axis_index("c")
        sid = lax.axis_index("s")
        eoff = cid * ECORE + sid * EPT

        pltpu.sync_copy(src_hbm.at[pl.ds(eoff, EPT)], src_v)
        pltpu.sync_copy(dst_hbm.at[pl.ds(eoff, EPT)], dst_v)

        for i in range(G // L):
            ones_v[pl.ds(i * L, L)] = jnp.full((L,), 1.0, jnp.float32)

        def zb(i, _):
            z_v[pl.ds(i * L, L)] = jnp.full((L,), 0.0, jnp.float32)
            return 0

        lax.fori_loop(0, TPW // L, zb, 0)
        tsl = pl.ds(sid * TPW, TPW)
        pltpu.sync_copy(z_v, dego_sh.at[tsl])
        pltpu.sync_copy(z_v, degi_sh.at[tsl])
        plsc.subcore_barrier()

        def chunk(j, _):
            g = pl.ds(j * G, G)
            pltpu.sync_copy(ones_v, dego_sh.at[src_v.at[g]], add=True)
            pltpu.sync_copy(ones_v, degi_sh.at[dst_v.at[g]], add=True)
            return 0

        lax.fori_loop(0, NCH, chunk, 0)
        plsc.subcore_barrier()

        osl = pl.ds(cid * NPAD + sid * TPW, TPW)
        pltpu.sync_copy(dego_sh.at[tsl], z_v)
        pltpu.sync_copy(z_v, do_hbm.at[osl])
        pltpu.sync_copy(degi_sh.at[tsl], z_v)
        pltpu.sync_copy(z_v, di_hbm.at[osl])

    return k1(src, dst)


def _k3_call(src, dst, dego_p, degi_p, weight_pad):
    @functools.partial(
        pl.kernel,
        out_type=(
            jax.ShapeDtypeStruct((2 * NPAD,), jnp.float32),
            jax.ShapeDtypeStruct((2 * NPAD,), jnp.float32),
        ),
        mesh=_mesh(),
        compiler_params=pltpu.CompilerParams(needs_layout_passes=False),
        scratch_types=[
            pltpu.VMEM((EPT,), jnp.int32),
            pltpu.VMEM((EPT,), jnp.int32),
            pltpu.VMEM((TPW,), jnp.float32),
            pltpu.VMEM((TPW,), jnp.float32),
            pltpu.VMEM((TPW,), jnp.float32),
            pltpu.VMEM((G,), jnp.float32),
            pltpu.VMEM((G,), jnp.float32),
            pltpu.VMEM_SHARED((NPAD,), jnp.float32),
            pltpu.VMEM_SHARED((NPAD,), jnp.float32),
            pltpu.VMEM_SHARED((NPAD,), jnp.float32),
            pltpu.VMEM_SHARED((NPAD,), jnp.float32),
            pltpu.SemaphoreType.DMA,
            pltpu.SemaphoreType.DMA,
        ],
    )
    def k3(src_hbm, dst_hbm, do_hbm, di_hbm, w_hbm, u_hbm, v_hbm,
           src_v, dst_v, b1_v, b2_v, b3_v, va_v, vb_v,
           nwt_sh, not_sh, u_sh, v_sh, sem1, sem2):
        cid = lax.axis_index("c")
        sid = lax.axis_index("s")
        eoff = cid * ECORE + sid * EPT

        pltpu.sync_copy(src_hbm.at[pl.ds(eoff, EPT)], src_v)
        pltpu.sync_copy(dst_hbm.at[pl.ds(eoff, EPT)], dst_v)

        tsl = pl.ds(sid * TPW, TPW)
        pltpu.sync_copy(do_hbm.at[pl.ds(sid * TPW, TPW)], b1_v)
        pltpu.sync_copy(do_hbm.at[pl.ds(NPAD + sid * TPW, TPW)], b2_v)
        pltpu.sync_copy(w_hbm.at[pl.ds(sid * TPW, TPW)], b3_v)

        def normb(i, _):
            sl = pl.ds(i * L, L)
            no = _rsqrt16(b1_v[sl] + b2_v[sl] + 1.0)
            b1_v[sl] = no
            b3_v[sl] = no * b3_v[sl]
            b2_v[sl] = jnp.full((L,), 0.0, jnp.float32)
            return 0

        lax.fori_loop(0, TPW // L, normb, 0)
        pltpu.sync_copy(b1_v, not_sh.at[tsl])
        pltpu.sync_copy(b3_v, nwt_sh.at[tsl])
        pltpu.sync_copy(b2_v, u_sh.at[tsl])
        pltpu.sync_copy(b2_v, v_sh.at[tsl])
        plsc.subcore_barrier()

        def chunk(j, _):
            g = pl.ds(j * G, G)
            cp1 = pltpu.async_copy(nwt_sh.at[src_v.at[g]], va_v, sem1)
            cp2 = pltpu.async_copy(not_sh.at[src_v.at[g]], vb_v, sem2)
            cp1.wait()
            cp2.wait()
            pltpu.sync_copy(va_v, u_sh.at[dst_v.at[g]], add=True)
            pltpu.sync_copy(vb_v, v_sh.at[dst_v.at[g]], add=True)
            return 0

        lax.fori_loop(0, NCH, chunk, 0)
        plsc.subcore_barrier()

        osl = pl.ds(cid * NPAD + sid * TPW, TPW)
        pltpu.sync_copy(u_sh.at[tsl], b1_v)
        pltpu.sync_copy(b1_v, u_hbm.at[osl])
        pltpu.sync_copy(v_sh.at[tsl], b2_v)
        pltpu.sync_copy(b2_v, v_hbm.at[osl])

    return k3(src, dst, dego_p, degi_p, weight_pad)


def _k4_call(u_p2, v_p2, do_p2, di_p2, w2, W_lin, b_lin2, W0, b02):
    def body(u0_ref, u1_ref, v0_ref, v1_ref, do0_ref, do1_ref,
             di0_ref, di1_ref, w_ref, wl_ref, bl_ref, w0_ref, b0_ref,
             m0_ref, m1_ref):
        r1 = jnp.dot(wl_ref[...], w0_ref[...],
                     preferred_element_type=jnp.float32)
        r2 = jnp.dot(bl_ref[...], w0_ref[...],
                     preferred_element_type=jnp.float32)
        no = lax.rsqrt(do0_ref[...] + do1_ref[...] + 1.0)
        ni = lax.rsqrt(di0_ref[...] + di1_ref[...] + 1.0)
        u = u0_ref[...] + u1_ref[...] + no * w_ref[...]
        v = v0_ref[...] + v1_ref[...] + no
        h0 = (ni * u) * r1 + (ni * v) * r2 + b0_ref[...]
        g0 = jnp.where(h0 >= 0, h0, _SLOPE * h0)
        m = g0 * no
        m0_ref[...] = m[:, :DH]
        m1_ref[...] = m[:, DH:]

    vec0 = pl.BlockSpec((BR, 1), lambda i: (i, 0))
    vec1 = pl.BlockSpec((BR, 1), lambda i: (NBLK + i, 0))
    full = lambda s: pl.BlockSpec(s, lambda i: (0, 0))
    return pl.pallas_call(
        body,
        grid=(NBLK,),
        in_specs=[vec0, vec1, vec0, vec1, vec0, vec1, vec0, vec1, vec0,
                  full((1, D)), full((1, D)), full((D, D)), full((1, D))],
        out_specs=[pl.BlockSpec((BR, DH), lambda i: (i, 0)),
                   pl.BlockSpec((BR, DH), lambda i: (i, 0))],
        out_shape=[jax.ShapeDtypeStruct((NPAD, DH), jnp.float32),
                   jax.ShapeDtypeStruct((NPAD, DH), jnp.float32)],
    )(u_p2, u_p2, v_p2, v_p2, do_p2, do_p2, di_p2, di_p2, w2,
      W_lin, b_lin2, W0, b02)


def _k5_call(m0, m1, src, dst):
    @functools.partial(
        pl.kernel,
        out_type=jax.ShapeDtypeStruct((4 * NPAD, DH), jnp.float32),
        mesh=_mesh(),
        compiler_params=pltpu.CompilerParams(needs_layout_passes=False),
        scratch_types=[
            pltpu.VMEM((EPT,), jnp.int32),
            pltpu.VMEM((EPT,), jnp.int32),
            pltpu.VMEM((G, DH), jnp.float32),
            pltpu.VMEM((G, DH), jnp.float32),
            pltpu.VMEM_SHARED((NPAD, DH), jnp.float32),
            pltpu.SemaphoreType.DMA,
            pltpu.SemaphoreType.DMA,
        ],
    )
    def k5(m0_hbm, m1_hbm, src_hbm, dst_hbm, agg_hbm,
           src_v, dst_v, rows0_v, rows1_v, acc_sh, sem0, sem1):
        cid = lax.axis_index("c")
        sid = lax.axis_index("s")
        eoff = cid * ECORE + sid * EPT

        pltpu.sync_copy(src_hbm.at[pl.ds(eoff, EPT)], src_v)
        pltpu.sync_copy(dst_hbm.at[pl.ds(eoff, EPT)], dst_v)

        for p, m_hbm in ((0, m0_hbm), (1, m1_hbm)):
            def zrow(i, _):
                for cc in range(DH // L):
                    rows0_v[i, pl.ds(cc * L, L)] = jnp.full(
                        (L,), 0.0, jnp.float32)
                return 0

            lax.fori_loop(0, G, zrow, 0)
            for q in range(TPW // G):
                pltpu.sync_copy(
                    rows0_v, acc_sh.at[pl.ds(sid * TPW + q * G, G)])
            plsc.subcore_barrier()

            cp0 = pltpu.async_copy(
                m_hbm.at[src_v.at[pl.ds(0, G)]], rows0_v, sem0)

            def pair(i, _):
                j0 = 2 * i
                j1 = 2 * i + 1
                cpb = pltpu.async_copy(
                    m_hbm.at[src_v.at[pl.ds(j1 * G, G)]], rows1_v, sem1)
                pltpu.make_async_copy(
                    m_hbm.at[src_v.at[pl.ds(j0 * G, G)]], rows0_v, sem0
                ).wait()
                pltpu.sync_copy(
                    rows0_v, acc_sh.at[dst_v.at[pl.ds(j0 * G, G)]], add=True)
                cpa = pltpu.async_copy(
                    m_hbm.at[src_v.at[pl.ds((j1 + 1) * G, G)]], rows0_v, sem0)
                cpb.wait()
                pltpu.sync_copy(
                    rows1_v, acc_sh.at[dst_v.at[pl.ds(j1 * G, G)]], add=True)
                return 0

            lax.fori_loop(0, (NCH - 1) // 2, pair, 0)
            pltpu.make_async_copy(
                m_hbm.at[src_v.at[pl.ds((NCH - 1) * G, G)]], rows0_v, sem0
            ).wait()
            pltpu.sync_copy(
                rows0_v, acc_sh.at[dst_v.at[pl.ds((NCH - 1) * G, G)]],
                add=True)
            plsc.subcore_barrier()

            for q in range(TPW // G):
                pltpu.sync_copy(
                    acc_sh.at[pl.ds(sid * TPW + q * G, G)], rows0_v)
                pltpu.sync_copy(
                    rows0_v,
                    agg_hbm.at[pl.ds(
                        (2 * p + cid) * NPAD + sid * TPW + q * G, G)])
            plsc.subcore_barrier()

    return k5(m0, m1, src, dst)


def _k6_call(agg4, m0, m1, do_p2, di_p2, W1, b12, W2, Wp, b22, bp2):
    def body(o00_ref, o01_ref, o10_ref, o11_ref, m0_ref, m1_ref,
             do0_ref, do1_ref, di0_ref, di1_ref,
             w1t_ref, w1b_ref, b1_ref, w2_ref, wp_ref, b2_ref, bp_ref,
             t_ref, c0_ref):
        no = lax.rsqrt(do0_ref[...] + do1_ref[...] + 1.0)
        ni = lax.rsqrt(di0_ref[...] + di1_ref[...] + 1.0)
        x0 = (o00_ref[...] + o01_ref[...] + m0_ref[...]) * ni
        x1 = (o10_ref[...] + o11_ref[...] + m1_ref[...]) * ni
        h1 = (jnp.dot(x0, w1t_ref[...], preferred_element_type=jnp.float32)
              + jnp.dot(x1, w1b_ref[...], preferred_element_type=jnp.float32)
              + b1_ref[...])
        g1 = jnp.where(h1 >= 0, h1, _SLOPE * h1)
        w2p = jnp.dot(w2_ref[...], wp_ref[...],
                      preferred_element_type=jnp.float32)
        q = jnp.dot(g1, w2p, preferred_element_type=jnp.float32)
        t_ref[...] = no * q

        @pl.when(pl.program_id(0) == 0)
        def _():
            c0 = (jnp.dot(b2_ref[...], wp_ref[...],
                          preferred_element_type=jnp.float32)[0, 0]
                  + bp_ref[0, 0])
            c0_ref[...] = jnp.full((1, L), c0, jnp.float32)

    vec0 = pl.BlockSpec((BR, 1), lambda i: (i, 0))
    vec1 = pl.BlockSpec((BR, 1), lambda i: (NBLK + i, 0))
    half = lambda k: pl.BlockSpec((BR, DH), lambda i, k=k: (k * NBLK + i, 0))
    mblk = pl.BlockSpec((BR, DH), lambda i: (i, 0))
    full = lambda s: pl.BlockSpec(s, lambda i: (0, 0))
    return pl.pallas_call(
        body,
        grid=(NBLK,),
        in_specs=[half(0), half(1), half(2), half(3), mblk, mblk,
                  vec0, vec1, vec0, vec1,
                  pl.BlockSpec((DH, D), lambda i: (0, 0)),
                  pl.BlockSpec((DH, D), lambda i: (1, 0)),
                  full((1, D)), full((D, D)), full((D, 1)),
                  full((1, D)), full((1, 1))],
        out_specs=[vec0, full((1, L))],
        out_shape=[jax.ShapeDtypeStruct((NPAD, 1), jnp.float32),
                   jax.ShapeDtypeStruct((1, L), jnp.float32)],
    )(agg4, agg4, agg4, agg4, m0, m1, do_p2, do_p2, di_p2, di_p2,
      W1, W1, b12, W2, Wp, b22, bp2)


def _k7_call(t, src, dst):
    @functools.partial(
        pl.kernel,
        out_type=jax.ShapeDtypeStruct((2 * NPAD,), jnp.float32),
        mesh=_mesh(),
        compiler_params=pltpu.CompilerParams(needs_layout_passes=False),
        scratch_types=[
            pltpu.VMEM((EPT,), jnp.int32),
            pltpu.VMEM((EPT,), jnp.int32),
            pltpu.VMEM((G,), jnp.float32),
            pltpu.VMEM((TPW,), jnp.float32),
            pltpu.VMEM_SHARED((NPAD,), jnp.float32),
            pltpu.VMEM_SHARED((NPAD,), jnp.float32),
            pltpu.SemaphoreType.DMA,
        ],
    )
    def k7(t_hbm, src_hbm, dst_hbm, s_hbm,
           src_v, dst_v, va_v, stage_v, tt_sh, sacc_sh, sem1):
        cid = lax.axis_index("c")
        sid = lax.axis_index("s")
        eoff = cid * ECORE + sid * EPT

        pltpu.sync_copy(src_hbm.at[pl.ds(eoff, EPT)], src_v)
        pltpu.sync_copy(dst_hbm.at[pl.ds(eoff, EPT)], dst_v)

        tsl = pl.ds(sid * TPW, TPW)
        pltpu.sync_copy(t_hbm.at[tsl], stage_v)
        pltpu.sync_copy(stage_v, tt_sh.at[tsl])

        def zb(i, _):
            stage_v[pl.ds(i * L, L)] = jnp.full((L,), 0.0, jnp.float32)
            return 0

        lax.fori_loop(0, TPW // L, zb, 0)
        pltpu.sync_copy(stage_v, sacc_sh.at[tsl])
        plsc.subcore_barrier()

        def chunk(j, _):
            g = pl.ds(j * G, G)
            pltpu.async_copy(tt_sh.at[src_v.at[g]], va_v, sem1).wait()
            pltpu.sync_copy(va_v, sacc_sh.at[dst_v.at[g]], add=True)
            return 0

        lax.fori_loop(0, NCH, chunk, 0)
        plsc.subcore_barrier()

        pltpu.sync_copy(sacc_sh.at[tsl], stage_v)
        pltpu.sync_copy(stage_v, s_hbm.at[pl.ds(cid * NPAD + sid * TPW, TPW)])

    return k7(t, src, dst)


def _k8_call(s_p2, t2, di_p2, c0b):
    def body(s0_ref, s1_ref, t_ref, di0_ref, di1_ref, c0_ref, out_ref):
        ni = lax.rsqrt(di0_ref[...] + di1_ref[...] + 1.0)
        out_ref[...] = ni * (s0_ref[...] + s1_ref[...] + t_ref[...]) \
            + c0_ref[0, 0]

    vec0 = pl.BlockSpec((BR, 1), lambda i: (i, 0))
    vec1 = pl.BlockSpec((BR, 1), lambda i: (NBLK + i, 0))
    return pl.pallas_call(
        body,
        grid=(NBLK,),
        in_specs=[vec0, vec1, vec0, vec0, vec1,
                  pl.BlockSpec((1, L), lambda i: (0, 0))],
        out_specs=vec0,
        out_shape=jax.ShapeDtypeStruct((NPAD, 1), jnp.float32),
    )(s_p2, s_p2, t2, di_p2, di_p2, c0b)


def kernel(weight, edge_index, W_lin, b_lin, W0, b0, W1, b1, W2, b2, Wp, bp):
    src = edge_index[0]
    dst = edge_index[1]
    weight_pad = jnp.pad(weight, (0, NPAD - N))

    dego_p, degi_p = _k1_call(src, dst)
    u_p, v_p = _k3_call(src, dst, dego_p, degi_p, weight_pad)

    to2 = lambda a: a.reshape(-1, 1)
    m0, m1 = _k4_call(to2(u_p), to2(v_p), to2(dego_p), to2(degi_p),
                      to2(weight_pad), W_lin, b_lin.reshape(1, D),
                      W0, b0.reshape(1, D))
    agg4 = _k5_call(m0, m1, src, dst)
    t2, c0b = _k6_call(agg4, m0, m1, to2(dego_p), to2(degi_p),
                       W1, b1.reshape(1, D), W2, Wp, b2.reshape(1, D),
                       bp.reshape(1, 1))
    s_p = _k7_call(t2.reshape(NPAD), src, dst)
    logits2 = _k8_call(to2(s_p), t2, to2(degi_p), c0b)
    return logits2[:N]


# speedup vs baseline: 14.6361x; 1.1394x over previous
import functools

import jax
import jax.numpy as jnp
from jax import lax
from jax.experimental import pallas as pl
from jax.experimental.pallas import tpu as pltpu, tpu_sc as plsc

N = 10000
D = 256
E = 160000

NPAD = 10240
NS = 16
L = 16
ECORE = E // 2
EPT = ECORE // NS
G = 40
NCH = EPT // G
TPW = NPAD // NS
DH = D // 2
BR = 2560
NBLK = NPAD // BR

_SLOPE = 0.01


def _mesh():
    return plsc.VectorSubcoreMesh(core_axis_name="c", subcore_axis_name="s")


def _rsqrt16(x):
    i = plsc.bitcast(x, jnp.int32)
    i = jnp.int32(0x5F3759DF) - lax.shift_right_logical(i, 1)
    y = plsc.bitcast(i, jnp.float32)
    for _ in range(3):
        y = y * (1.5 - 0.5 * x * y * y)
    return y


def _k1_call(src, dst):
    @functools.partial(
        pl.kernel,
        out_type=(
            jax.ShapeDtypeStruct((2 * NPAD,), jnp.float32),
            jax.ShapeDtypeStruct((2 * NPAD,), jnp.float32),
        ),
        mesh=_mesh(),
        compiler_params=pltpu.CompilerParams(needs_layout_passes=False),
        scratch_types=[
            pltpu.VMEM((EPT,), jnp.int32),
            pltpu.VMEM((EPT,), jnp.int32),
            pltpu.VMEM((G,), jnp.float32),
            pltpu.VMEM((TPW,), jnp.float32),
            pltpu.VMEM_SHARED((NPAD,), jnp.float32),
            pltpu.VMEM_SHARED((NPAD,), jnp.float32),
        ],
    )
    def k1(src_hbm, dst_hbm, do_hbm, di_hbm,
           src_v, dst_v, ones_v, z_v, dego_sh, degi_sh):
        cid = lax.axis_index("c")
        sid = lax.axis_index("s")
        eoff = cid * ECORE + sid * EPT

        pltpu.sync_copy(src_hbm.at[pl.ds(eoff, EPT)], src_v)
        pltpu.sync_copy(dst_hbm.at[pl.ds(eoff, EPT)], dst_v)

        for i in range(G // L):
            ones_v[pl.ds(i * L, L)] = jnp.full((L,), 1.0, jnp.float32)

        def zb(i, _):
            z_v[pl.ds(i * L, L)] = jnp.full((L,), 0.0, jnp.float32)
            return 0

        lax.fori_loop(0, TPW // L, zb, 0)
        tsl = pl.ds(sid * TPW, TPW)
        pltpu.sync_copy(z_v, dego_sh.at[tsl])
        pltpu.sync_copy(z_v, degi_sh.at[tsl])
        plsc.subcore_barrier()

        def chunk(j, _):
            g = pl.ds(j * G, G)
            pltpu.sync_copy(ones_v, dego_sh.at[src_v.at[g]], add=True)
            pltpu.sync_copy(ones_v, degi_sh.at[dst_v.at[g]], add=True)
            return 0

        lax.fori_loop(0, NCH, chunk, 0)
        plsc.subcore_barrier()

        osl = pl.ds(cid * NPAD + sid * TPW, TPW)
        pltpu.sync_copy(dego_sh.at[tsl], z_v)
        pltpu.sync_copy(z_v, do_hbm.at[osl])
        pltpu.sync_copy(degi_sh.at[tsl], z_v)
        pltpu.sync_copy(z_v, di_hbm.at[osl])

    return k1(src, dst)


def _k3_call(src, dst, dego_p, degi_p, weight_pad):
    @functools.partial(
        pl.kernel,
        out_type=(
            jax.ShapeDtypeStruct((2 * NPAD,), jnp.float32),
            jax.ShapeDtypeStruct((2 * NPAD,), jnp.float32),
        ),
        mesh=_mesh(),
        compiler_params=pltpu.CompilerParams(needs_layout_passes=False),
        scratch_types=[
            pltpu.VMEM((EPT,), jnp.int32),
            pltpu.VMEM((EPT,), jnp.int32),
            pltpu.VMEM((TPW,), jnp.float32),
            pltpu.VMEM((TPW,), jnp.float32),
            pltpu.VMEM((TPW,), jnp.float32),
            pltpu.VMEM((G,), jnp.float32),
            pltpu.VMEM((G,), jnp.float32),
            pltpu.VMEM_SHARED((NPAD,), jnp.float32),
            pltpu.VMEM_SHARED((NPAD,), jnp.float32),
            pltpu.VMEM_SHARED((NPAD,), jnp.float32),
            pltpu.VMEM_SHARED((NPAD,), jnp.float32),
            pltpu.SemaphoreType.DMA,
            pltpu.SemaphoreType.DMA,
        ],
    )
    def k3(src_hbm, dst_hbm, do_hbm, di_hbm, w_hbm, u_hbm, v_hbm,
           src_v, dst_v, b1_v, b2_v, b3_v, va_v, vb_v,
           nwt_sh, not_sh, u_sh, v_sh, sem1, sem2):
        cid = lax.axis_index("c")
        sid = lax.axis_index("s")
        eoff = cid * ECORE + sid * EPT

        pltpu.sync_copy(src_hbm.at[pl.ds(eoff, EPT)], src_v)
        pltpu.sync_copy(dst_hbm.at[pl.ds(eoff, EPT)], dst_v)

        tsl = pl.ds(sid * TPW, TPW)
        pltpu.sync_copy(do_hbm.at[pl.ds(sid * TPW, TPW)], b1_v)
        pltpu.sync_copy(do_hbm.at[pl.ds(NPAD + sid * TPW, TPW)], b2_v)
        pltpu.sync_copy(w_hbm.at[pl.ds(sid * TPW, TPW)], b3_v)

        def normb(i, _):
            sl = pl.ds(i * L, L)
            no = _rsqrt16(b1_v[sl] + b2_v[sl] + 1.0)
            b1_v[sl] = no
            b3_v[sl] = no * b3_v[sl]
            b2_v[sl] = jnp.full((L,), 0.0, jnp.float32)
            return 0

        lax.fori_loop(0, TPW // L, normb, 0)
        pltpu.sync_copy(b1_v, not_sh.at[tsl])
        pltpu.sync_copy(b3_v, nwt_sh.at[tsl])
        pltpu.sync_copy(b2_v, u_sh.at[tsl])
        pltpu.sync_copy(b2_v, v_sh.at[tsl])
        plsc.subcore_barrier()

        def chunk(j, _):
            g = pl.ds(j * G, G)
            cp1 = pltpu.async_copy(nwt_sh.at[src_v.at[g]], va_v, sem1)
            cp2 = pltpu.async_copy(not_sh.at[src_v.at[g]], vb_v, sem2)
            cp1.wait()
            cp2.wait()
            pltpu.sync_copy(va_v, u_sh.at[dst_v.at[g]], add=True)
            pltpu.sync_copy(vb_v, v_sh.at[dst_v.at[g]], add=True)
            return 0

        lax.fori_loop(0, NCH, chunk, 0)
        plsc.subcore_barrier()

        osl = pl.ds(cid * NPAD + sid * TPW, TPW)
        pltpu.sync_copy(u_sh.at[tsl], b1_v)
        pltpu.sync_copy(b1_v, u_hbm.at[osl])
        pltpu.sync_copy(v_sh.at[tsl], b2_v)
        pltpu.sync_copy(b2_v, v_hbm.at[osl])

    return k3(src, dst, dego_p, degi_p, weight_pad)


def _k4_call(u_p2, v_p2, do_p2, di_p2, w2, W_lin, b_lin2, W0, b02):
    def body(u0_ref, u1_ref, v0_ref, v1_ref, do0_ref, do1_ref,
             di0_ref, di1_ref, w_ref, wl_ref, bl_ref, w0_ref, b0_ref,
             m0_ref, m1_ref):
        r1 = jnp.dot(wl_ref[...], w0_ref[...],
                     preferred_element_type=jnp.float32)
        r2 = jnp.dot(bl_ref[...], w0_ref[...],
                     preferred_element_type=jnp.float32)
        no = lax.rsqrt(do0_ref[...] + do1_ref[...] + 1.0)
        ni = lax.rsqrt(di0_ref[...] + di1_ref[...] + 1.0)
        u = u0_ref[...] + u1_ref[...] + no * w_ref[...]
        v = v0_ref[...] + v1_ref[...] + no
        h0 = (ni * u) * r1 + (ni * v) * r2 + b0_ref[...]
        g0 = jnp.where(h0 >= 0, h0, _SLOPE * h0)
        m = g0 * no
        m0_ref[...] = m[:, :DH]
        m1_ref[...] = m[:, DH:]

    vec0 = pl.BlockSpec((BR, 1), lambda i: (i, 0))
    vec1 = pl.BlockSpec((BR, 1), lambda i: (NBLK + i, 0))
    full = lambda s: pl.BlockSpec(s, lambda i: (0, 0))
    return pl.pallas_call(
        body,
        grid=(NBLK,),
        in_specs=[vec0, vec1, vec0, vec1, vec0, vec1, vec0, vec1, vec0,
                  full((1, D)), full((1, D)), full((D, D)), full((1, D))],
        out_specs=[pl.BlockSpec((BR, DH), lambda i: (i, 0)),
                   pl.BlockSpec((BR, DH), lambda i: (i, 0))],
        out_shape=[jax.ShapeDtypeStruct((NPAD, DH), jnp.float32),
                   jax.ShapeDtypeStruct((NPAD, DH), jnp.float32)],
    )(u_p2, u_p2, v_p2, v_p2, do_p2, do_p2, di_p2, di_p2, w2,
      W_lin, b_lin2, W0, b02)


def _k5_call(m0, m1, src, dst):
    @functools.partial(
        pl.kernel,
        out_type=jax.ShapeDtypeStruct((4 * NPAD, DH), jnp.float32),
        mesh=_mesh(),
        compiler_params=pltpu.CompilerParams(needs_layout_passes=False),
        scratch_types=[
            pltpu.VMEM((EPT,), jnp.int32),
            pltpu.VMEM((EPT,), jnp.int32),
            pltpu.VMEM((G, DH), jnp.float32),
            pltpu.VMEM((G, DH), jnp.float32),
            pltpu.VMEM_SHARED((NPAD, DH), jnp.float32),
            pltpu.SemaphoreType.DMA,
            pltpu.SemaphoreType.DMA,
        ],
    )
    def k5(m0_hbm, m1_hbm, src_hbm, dst_hbm, agg_hbm,
           src_v, dst_v, rows0_v, rows1_v, acc_sh, sem0, sem1):
        cid = lax.axis_index("c")
        sid = lax.axis_index("s")
        eoff = cid * ECORE + sid * EPT

        pltpu.sync_copy(src_hbm.at[pl.ds(eoff, EPT)], src_v)
        pltpu.sync_copy(dst_hbm.at[pl.ds(eoff, EPT)], dst_v)

        for p, m_hbm in ((0, m0_hbm), (1, m1_hbm)):
            def zrow(i, _):
                for cc in range(DH // L):
                    rows0_v[i, pl.ds(cc * L, L)] = jnp.full(
                        (L,), 0.0, jnp.float32)
                return 0

            lax.fori_loop(0, G, zrow, 0)
            for q in range(TPW // G):
                pltpu.sync_copy(
                    rows0_v, acc_sh.at[pl.ds(sid * TPW + q * G, G)])
            plsc.subcore_barrier()

            cp0 = pltpu.async_copy(
                m_hbm.at[src_v.at[pl.ds(0, G)]], rows0_v, sem0)

            def pair(i, _):
                j0 = 2 * i
                j1 = 2 * i + 1
                cpb = pltpu.async_copy(
                    m_hbm.at[src_v.at[pl.ds(j1 * G, G)]], rows1_v, sem1)
                pltpu.make_async_copy(
                    m_hbm.at[src_v.at[pl.ds(j0 * G, G)]], rows0_v, sem0
                ).wait()
                pltpu.sync_copy(
                    rows0_v, acc_sh.at[dst_v.at[pl.ds(j0 * G, G)]], add=True)
                cpa = pltpu.async_copy(
                    m_hbm.at[src_v.at[pl.ds((j1 + 1) * G, G)]], rows0_v, sem0)
                cpb.wait()
                pltpu.sync_copy(
                    rows1_v, acc_sh.at[dst_v.at[pl.ds(j1 * G, G)]], add=True)
                return 0

            lax.fori_loop(0, (NCH - 1) // 2, pair, 0)
            pltpu.make_async_copy(
                m_hbm.at[src_v.at[pl.ds((NCH - 1) * G, G)]], rows0_v, sem0
            ).wait()
            pltpu.sync_copy(
                rows0_v, acc_sh.at[dst_v.at[pl.ds((NCH - 1) * G, G)]],
                add=True)
            plsc.subcore_barrier()

            for q in range(TPW // G):
                pltpu.sync_copy(
                    acc_sh.at[pl.ds(sid * TPW + q * G, G)], rows0_v)
                pltpu.sync_copy(
                    rows0_v,
                    agg_hbm.at[pl.ds(
                        (2 * p + cid) * NPAD + sid * TPW + q * G, G)])
            plsc.subcore_barrier()

    return k5(m0, m1, src, dst)


def _k6_call(agg4, m0, m1, do_p2, di_p2, W1, b12, W2, Wp, b22, bp2):
    def body(o00_ref, o01_ref, o10_ref, o11_ref, m0_ref, m1_ref,
             do0_ref, do1_ref, di0_ref, di1_ref,
             w1t_ref, w1b_ref, b1_ref, w2_ref, wp_ref, b2_ref, bp_ref,
             t_ref, c0_ref):
        no = lax.rsqrt(do0_ref[...] + do1_ref[...] + 1.0)
        ni = lax.rsqrt(di0_ref[...] + di1_ref[...] + 1.0)
        x0 = (o00_ref[...] + o01_ref[...] + m0_ref[...]) * ni
        x1 = (o10_ref[...] + o11_ref[...] + m1_ref[...]) * ni
        h1 = (jnp.dot(x0, w1t_ref[...], preferred_element_type=jnp.float32)
              + jnp.dot(x1, w1b_ref[...], preferred_element_type=jnp.float32)
              + b1_ref[...])
        g1 = jnp.where(h1 >= 0, h1, _SLOPE * h1)
        w2p = jnp.dot(w2_ref[...], wp_ref[...],
                      preferred_element_type=jnp.float32)
        q = jnp.dot(g1, w2p, preferred_element_type=jnp.float32)
        t_ref[...] = no * q

        @pl.when(pl.program_id(0) == 0)
        def _():
            c0 = (jnp.dot(b2_ref[...], wp_ref[...],
                          preferred_element_type=jnp.float32)[0, 0]
                  + bp_ref[0, 0])
            c0_ref[...] = jnp.full((1, L), c0, jnp.float32)

    vec0 = pl.BlockSpec((BR, 1), lambda i: (i, 0))
    vec1 = pl.BlockSpec((BR, 1), lambda i: (NBLK + i, 0))
    half = lambda k: pl.BlockSpec((BR, DH), lambda i, k=k: (k * NBLK + i, 0))
    mblk = pl.BlockSpec((BR, DH), lambda i: (i, 0))
    full = lambda s: pl.BlockSpec(s, lambda i: (0, 0))
    return pl.pallas_call(
        body,
        grid=(NBLK,),
        in_specs=[half(0), half(1), half(2), half(3), mblk, mblk,
                  vec0, vec1, vec0, vec1,
                  pl.BlockSpec((DH, D), lambda i: (0, 0)),
                  pl.BlockSpec((DH, D), lambda i: (1, 0)),
                  full((1, D)), full((D, D)), full((D, 1)),
                  full((1, D)), full((1, 1))],
        out_specs=[vec0, full((1, L))],
        out_shape=[jax.ShapeDtypeStruct((NPAD, 1), jnp.float32),
                   jax.ShapeDtypeStruct((1, L), jnp.float32)],
    )(agg4, agg4, agg4, agg4, m0, m1, do_p2, do_p2, di_p2, di_p2,
      W1, W1, b12, W2, Wp, b22, bp2)


def _k7_call(t, src, dst):
    @functools.partial(
        pl.kernel,
        out_type=jax.ShapeDtypeStruct((2 * NPAD,), jnp.float32),
        mesh=_mesh(),
        compiler_params=pltpu.CompilerParams(needs_layout_passes=False),
        scratch_types=[
            pltpu.VMEM((EPT,), jnp.int32),
            pltpu.VMEM((EPT,), jnp.int32),
            pltpu.VMEM((G,), jnp.float32),
            pltpu.VMEM((TPW,), jnp.float32),
            pltpu.VMEM_SHARED((NPAD,), jnp.float32),
            pltpu.VMEM_SHARED((NPAD,), jnp.float32),
            pltpu.SemaphoreType.DMA,
        ],
    )
    def k7(t_hbm, src_hbm, dst_hbm, s_hbm,
           src_v, dst_v, va_v, stage_v, tt_sh, sacc_sh, sem1):
        cid = lax.axis_index("c")
        sid = lax.axis_index("s")
        eoff = cid * ECORE + sid * EPT

        pltpu.sync_copy(src_hbm.at[pl.ds(eoff, EPT)], src_v)
        pltpu.sync_copy(dst_hbm.at[pl.ds(eoff, EPT)], dst_v)

        tsl = pl.ds(sid * TPW, TPW)
        pltpu.sync_copy(t_hbm.at[tsl], stage_v)
        pltpu.sync_copy(stage_v, tt_sh.at[tsl])

        def zb(i, _):
            stage_v[pl.ds(i * L, L)] = jnp.full((L,), 0.0, jnp.float32)
            return 0

        lax.fori_loop(0, TPW // L, zb, 0)
        pltpu.sync_copy(stage_v, sacc_sh.at[tsl])
        plsc.subcore_barrier()

        def chunk(j, _):
            g = pl.ds(j * G, G)
            pltpu.async_copy(tt_sh.at[src_v.at[g]], va_v, sem1).wait()
            pltpu.sync_copy(va_v, sacc_sh.at[dst_v.at[g]], add=True)
            return 0

        lax.fori_loop(0, NCH, chunk, 0)
        plsc.subcore_barrier()

        pltpu.sync_copy(sacc_sh.at[tsl], stage_v)
        pltpu.sync_copy(stage_v, s_hbm.at[pl.ds(cid * NPAD + sid * TPW, TPW)])

    return k7(t, src, dst)


def _k8_call(s_p2, t2, di_p2, c0b):
    def body(s0_ref, s1_ref, t_ref, di0_ref, di1_ref, c0_ref, out_ref):
        ni = lax.rsqrt(di0_ref[...] + di1_ref[...] + 1.0)
        out_ref[...] = ni * (s0_ref[...] + s1_ref[...] + t_ref[...]) \
            + c0_ref[0, 0]

    vec0 = pl.BlockSpec((BR, 1), lambda i: (i, 0))
    vec1 = pl.BlockSpec((BR, 1), lambda i: (NBLK + i, 0))
    return pl.pallas_call(
        body,
        grid=(NBLK,),
        in_specs=[vec0, vec1, vec0, vec0, vec1,
                  pl.BlockSpec((1, L), lambda i: (0, 0))],
        out_specs=vec0,
        out_shape=jax.ShapeDtypeStruct((NPAD, 1), jnp.float32),
    )(s_p2, s_p2, t2, di_p2, di_p2, c0b)


def kernel(weight, edge_index, W_lin, b_lin, W0, b0, W1, b1, W2, b2, Wp, bp):
    src = edge_index[0]
    dst = edge_index[1]
    weight_pad = jnp.pad(weight, (0, NPAD - N))

    dego_p, degi_p = _k1_call(src, dst)
    u_p, v_p = _k3_call(src, dst, dego_p, degi_p, weight_pad)

    to2 = lambda a: a.reshape(-1, 1)
    m0, m1 = _k4_call(to2(u_p), to2(v_p), to2(dego_p), to2(degi_p),
                      to2(weight_pad), W_lin, b_lin.reshape(1, D),
                      W0, b0.reshape(1, D))
    agg4 = _k5_call(m0, m1, src, dst)
    t2, c0b = _k6_call(agg4, m0, m1, to2(dego_p), to2(degi_p),
                       W1, b1.reshape(1, D), W2, Wp, b2.reshape(1, D),
                       bp.reshape(1, 1))
    s_p = _k7_call(t2.reshape(NPAD), src, dst)
    logits2 = _k8_call(to2(s_p), t2, to2(degi_p), c0b)
    return logits2[:N]
